# baseline (speedup 1.0000x reference)
import jax
import jax.numpy as jnp
from jax import lax
from jax.experimental import pallas as pl
from jax.experimental.pallas import tpu as pltpu
from jax.experimental.pallas import tpu_sc as plsc

NC, NS, L = 2, 16, 16
NW = NC * NS

B, S, D = 4, 2048, 4096
PW = S // NW
P = 8
NCHUNK = PW // P
CH = D // 2
NP = NCHUNK * 2
T = NP * B


def _sc_body(x_hbm, pos_hbm, idx_hbm, out_hbm, idx_v, pos_v, xin_v, xout_v,
             sem_g, sem_in, sem_out):
    wid = lax.axis_index("s") * NC + lax.axis_index("c")
    base = wid * PW

    pltpu.sync_copy(idx_hbm.at[pl.ds(base, PW)], idx_v)

    def gather(p):
        c = p // 2
        h = p - c * 2
        return pltpu.make_async_copy(
            pos_hbm.at[idx_v.at[pl.ds(c * P, P)], pl.ds(h * CH, CH)],
            pos_v.at[lax.rem(p, 2)], sem_g)

    def split(t):
        p = t // B
        b = t - p * B
        c = p // 2
        h = p - c * 2
        return p, b, c, h

    def in_copy(t):
        p, b, c, h = split(t)
        return pltpu.make_async_copy(
            x_hbm.at[b, pl.ds(base + c * P, P), pl.ds(h * CH, CH)],
            xin_v.at[lax.rem(t, 2)], sem_in)

    def out_copy(t):
        p, b, c, h = split(t)
        return pltpu.make_async_copy(
            xout_v.at[lax.rem(t, 2)],
            out_hbm.at[b, pl.ds(base + c * P, P), pl.ds(h * CH, CH)],
            sem_out)

    gather(0).start()
    in_copy(0).start()
    in_copy(1).start()

    def step(t, carry):
        p, b, c, h = split(t)
        tb = lax.rem(t, 2)
        pb = lax.rem(p, 2)

        @pl.when(b == 0)
        def _():
            gather(p).wait()

        @pl.when(jnp.logical_and(b == 0, p + 1 < NP))
        def _():
            gather(p + 1).start()

        in_copy(t).wait()

        @pl.when(t >= 2)
        def _():
            out_copy(t - 2).wait()

        @plsc.parallel_loop(0, CH // L, unroll=8)
        def add_vecs(i):
            s = pl.ds(i * L, L)
            for r in range(P):
                xout_v[tb, r, s] = xin_v[tb, r, s] + pos_v[pb, r, s]

        out_copy(t).start()

        @pl.when(t + 2 < T)
        def _():
            in_copy(t + 2).start()

        return carry

    lax.fori_loop(0, T, step, 0)

    out_copy(T - 2).wait()
    out_copy(T - 1).wait()


def kernel(x, pos_emb, block_indices):
    idx = block_indices.astype(jnp.int32)
    k = pl.kernel(
        _sc_body,
        out_type=jax.ShapeDtypeStruct((B, S, D), jnp.float32),
        mesh=plsc.VectorSubcoreMesh(
            core_axis_name="c", subcore_axis_name="s",
            num_cores=NC, num_subcores=NS),
        scratch_types=[
            pltpu.VMEM((PW,), jnp.int32),
            pltpu.VMEM((2, P, CH), jnp.float32),
            pltpu.VMEM((2, P, CH), jnp.float32),
            pltpu.VMEM((2, P, CH), jnp.float32),
            pltpu.SemaphoreType.DMA,
            pltpu.SemaphoreType.DMA,
            pltpu.SemaphoreType.DMA,
        ],
    )
    return k(x, pos_emb, idx)


# speedup vs baseline: 1.0647x; 1.0647x over previous
import jax
import jax.numpy as jnp
from jax import lax
from jax.experimental import pallas as pl
from jax.experimental.pallas import tpu as pltpu
from jax.experimental.pallas import tpu_sc as plsc

NC, NS, L = 2, 16, 16
NW = NC * NS

B, S, D = 4, 2048, 4096
PW = S // NW
P = 8
NCHUNK = PW // P
CH = D // 2
NP = NCHUNK * 2
T = NP * B


def _sc_body(x_hbm, pos_hbm, idx_hbm, out_hbm, idx_v, pos_v, xin_v, xout_v,
             sem_g, sem_in, sem_out):
    wid = lax.axis_index("s") * NC + lax.axis_index("c")
    base = wid * PW

    pltpu.sync_copy(idx_hbm.at[pl.ds(base, PW)], idx_v)

    def gather(p):
        c = p // 2
        h = p - c * 2
        return pltpu.make_async_copy(
            pos_hbm.at[idx_v.at[pl.ds(c * P, P)], pl.ds(h * CH, CH)],
            pos_v.at[lax.rem(p, 2)], sem_g)

    def split(t):
        p = t // B
        b = t - p * B
        c = p // 2
        h = p - c * 2
        return p, b, c, h

    def in_copy(t):
        p, b, c, h = split(t)
        return pltpu.make_async_copy(
            x_hbm.at[b, pl.ds(base + c * P, P), pl.ds(h * CH, CH)],
            xin_v.at[lax.rem(t, 3)], sem_in)

    def out_copy(t):
        p, b, c, h = split(t)
        return pltpu.make_async_copy(
            xout_v.at[lax.rem(t, 2)],
            out_hbm.at[b, pl.ds(base + c * P, P), pl.ds(h * CH, CH)],
            sem_out)

    gather(0).start()
    in_copy(0).start()
    in_copy(1).start()
    in_copy(2).start()

    def step(t, carry):
        p, b, c, h = split(t)
        ti = lax.rem(t, 3)
        to = lax.rem(t, 2)
        pb = lax.rem(p, 2)

        @pl.when(b == 0)
        def _():
            gather(p).wait()

        @pl.when(jnp.logical_and(b == 0, p + 1 < NP))
        def _():
            gather(p + 1).start()

        in_copy(t).wait()

        @pl.when(t + 3 < T)
        def _():
            in_copy(t + 3).start()

        @pl.when(t >= 2)
        def _():
            out_copy(t - 2).wait()

        @plsc.parallel_loop(0, CH // L, unroll=8)
        def add_vecs(i):
            s = pl.ds(i * L, L)
            for r in range(P):
                xout_v[to, r, s] = xin_v[ti, r, s] + pos_v[pb, r, s]

        out_copy(t).start()

        return carry

    lax.fori_loop(0, T, step, 0)

    out_copy(T - 2).wait()
    out_copy(T - 1).wait()


def kernel(x, pos_emb, block_indices):
    idx = block_indices.astype(jnp.int32)
    k = pl.kernel(
        _sc_body,
        out_type=jax.ShapeDtypeStruct((B, S, D), jnp.float32),
        mesh=plsc.VectorSubcoreMesh(
            core_axis_name="c", subcore_axis_name="s",
            num_cores=NC, num_subcores=NS),
        scratch_types=[
            pltpu.VMEM((PW,), jnp.int32),
            pltpu.VMEM((2, P, CH), jnp.float32),
            pltpu.VMEM((3, P, CH), jnp.float32),
            pltpu.VMEM((2, P, CH), jnp.float32),
            pltpu.SemaphoreType.DMA,
            pltpu.SemaphoreType.DMA,
            pltpu.SemaphoreType.DMA,
        ],
    )
    return k(x, pos_emb, idx)
